# baseline (speedup 1.0000x reference)
import jax
import jax.numpy as jnp
from jax.experimental import pallas as pl
from jax.experimental.pallas import tpu as pltpu

N = 10000
NFEAT = 128
NHID = 16
NCLASS = 8

TI = 200
NI = N // TI


def _proj1_body(x_ref, w1_ref, s1_ref):
    s1_ref[...] = jnp.dot(x_ref[...], w1_ref[...],
                          preferred_element_type=jnp.float32)


def _pass1_body(adj_ref, s1_ref, w2_ref, b1_ref, s2_ref):
    h = jnp.maximum(jnp.dot(adj_ref[...], s1_ref[...],
                            preferred_element_type=jnp.float32)
                    + b1_ref[...], 0.0)
    s2_ref[...] = jnp.dot(h, w2_ref[...], preferred_element_type=jnp.float32)


def _pass2_body(adj_ref, s2_ref, b2_ref, o_ref):
    z = jnp.dot(adj_ref[...], s2_ref[...],
                preferred_element_type=jnp.float32) + b2_ref[...]
    m = jnp.max(z, axis=1, keepdims=True)
    lse = jnp.log(jnp.sum(jnp.exp(z - m), axis=1, keepdims=True)) + m
    o_ref[...] = z - lse


def kernel(x, adj, W1, b1, W2, b2):
    b1r = b1.reshape(1, NHID)
    b2r = b2.reshape(1, NCLASS)

    s1 = pl.pallas_call(
        _proj1_body,
        out_shape=jax.ShapeDtypeStruct((N, NHID), jnp.float32),
    )(x, W1)

    s2 = pl.pallas_call(
        _pass1_body,
        grid=(NI,),
        in_specs=[
            pl.BlockSpec((TI, N), lambda i: (i, 0)),
            pl.BlockSpec((N, NHID), lambda i: (0, 0)),
            pl.BlockSpec((NHID, NCLASS), lambda i: (0, 0)),
            pl.BlockSpec((1, NHID), lambda i: (0, 0)),
        ],
        out_specs=pl.BlockSpec((TI, NCLASS), lambda i: (i, 0)),
        out_shape=jax.ShapeDtypeStruct((N, NCLASS), jnp.float32),
        compiler_params=pltpu.CompilerParams(
            dimension_semantics=("parallel",)),
    )(adj, s1, W2, b1r)

    out = pl.pallas_call(
        _pass2_body,
        grid=(NI,),
        in_specs=[
            pl.BlockSpec((TI, N), lambda i: (i, 0)),
            pl.BlockSpec((N, NCLASS), lambda i: (0, 0)),
            pl.BlockSpec((1, NCLASS), lambda i: (0, 0)),
        ],
        out_specs=pl.BlockSpec((TI, NCLASS), lambda i: (i, 0)),
        out_shape=jax.ShapeDtypeStruct((N, NCLASS), jnp.float32),
        compiler_params=pltpu.CompilerParams(
            dimension_semantics=("parallel",)),
    )(adj, s2, b2r)

    return out


# speedup vs baseline: 1.0320x; 1.0320x over previous
import jax
import jax.numpy as jnp
from jax.experimental import pallas as pl
from jax.experimental.pallas import tpu as pltpu

N = 10000
NFEAT = 128
NHID = 16
NCLASS = 8

TI = 200
NI = N // TI
NSTEPS = 1 + 2 * NI


def _adj_index(g):
    return (jnp.where(g == 0, 0, (g - 1) % NI), 0)


def _out_index(g):
    return (jnp.where(g <= NI, 0, g - NI - 1), 0)


def _body(adj_ref, x_ref, w1_ref, w2_ref, b1_ref, b2_ref, o_ref,
          s1_ref, s2_ref):
    g = pl.program_id(0)

    @pl.when(g == 0)
    def _():
        s1_ref[...] = jnp.dot(x_ref[...], w1_ref[...],
                              preferred_element_type=jnp.float32)

    @pl.when((g >= 1) & (g <= NI))
    def _():
        h = jnp.maximum(jnp.dot(adj_ref[...], s1_ref[...],
                                preferred_element_type=jnp.float32)
                        + b1_ref[...], 0.0)
        s2_ref[pl.ds((g - 1) * TI, TI), :] = jnp.dot(
            h, w2_ref[...], preferred_element_type=jnp.float32)

    @pl.when(g > NI)
    def _():
        z = jnp.dot(adj_ref[...], s2_ref[...],
                    preferred_element_type=jnp.float32) + b2_ref[...]
        m = jnp.max(z, axis=1, keepdims=True)
        lse = jnp.log(jnp.sum(jnp.exp(z - m), axis=1, keepdims=True)) + m
        o_ref[...] = z - lse


def kernel(x, adj, W1, b1, W2, b2):
    return pl.pallas_call(
        _body,
        grid=(NSTEPS,),
        in_specs=[
            pl.BlockSpec((TI, N), _adj_index),
            pl.BlockSpec((N, NFEAT), lambda g: (0, 0)),
            pl.BlockSpec((NFEAT, NHID), lambda g: (0, 0)),
            pl.BlockSpec((NHID, NCLASS), lambda g: (0, 0)),
            pl.BlockSpec((1, NHID), lambda g: (0, 0)),
            pl.BlockSpec((1, NCLASS), lambda g: (0, 0)),
        ],
        out_specs=pl.BlockSpec((TI, NCLASS), _out_index),
        out_shape=jax.ShapeDtypeStruct((N, NCLASS), jnp.float32),
        scratch_shapes=[
            pltpu.VMEM((N, NHID), jnp.float32),
            pltpu.VMEM((N, NCLASS), jnp.float32),
        ],
        compiler_params=pltpu.CompilerParams(
            dimension_semantics=("arbitrary",)),
    )(adj, x, W1, W2, b1.reshape(1, NHID), b2.reshape(1, NCLASS))


# speedup vs baseline: 1.0724x; 1.0391x over previous
import jax
import jax.numpy as jnp
from jax.experimental import pallas as pl
from jax.experimental.pallas import tpu as pltpu

N = 10000
NFEAT = 128
NHID = 16
NCLASS = 8

TI = 400
NI = N // TI
NSTEPS = 1 + 2 * NI


def _adj_index(g):
    return (jnp.where(g == 0, 0, (g - 1) % NI), 0)


def _out_index(g):
    return (jnp.where(g <= NI, 0, g - NI - 1), 0)


def _body(adj_ref, x_ref, w1_ref, w2_ref, b1_ref, b2_ref, o_ref,
          s1_ref, s2_ref):
    g = pl.program_id(0)

    @pl.when(g == 0)
    def _():
        s1_ref[...] = jnp.dot(x_ref[...], w1_ref[...],
                              preferred_element_type=jnp.float32)

    @pl.when((g >= 1) & (g <= NI))
    def _():
        h = jnp.maximum(jnp.dot(adj_ref[...], s1_ref[...],
                                preferred_element_type=jnp.float32)
                        + b1_ref[...], 0.0)
        s2_ref[pl.ds((g - 1) * TI, TI), :] = jnp.dot(
            h, w2_ref[...], preferred_element_type=jnp.float32)

    @pl.when(g > NI)
    def _():
        z = jnp.dot(adj_ref[...], s2_ref[...],
                    preferred_element_type=jnp.float32) + b2_ref[...]
        m = jnp.max(z, axis=1, keepdims=True)
        lse = jnp.log(jnp.sum(jnp.exp(z - m), axis=1, keepdims=True)) + m
        o_ref[...] = z - lse


def kernel(x, adj, W1, b1, W2, b2):
    return pl.pallas_call(
        _body,
        grid=(NSTEPS,),
        in_specs=[
            pl.BlockSpec((TI, N), _adj_index),
            pl.BlockSpec((N, NFEAT), lambda g: (0, 0)),
            pl.BlockSpec((NFEAT, NHID), lambda g: (0, 0)),
            pl.BlockSpec((NHID, NCLASS), lambda g: (0, 0)),
            pl.BlockSpec((1, NHID), lambda g: (0, 0)),
            pl.BlockSpec((1, NCLASS), lambda g: (0, 0)),
        ],
        out_specs=pl.BlockSpec((TI, NCLASS), _out_index),
        out_shape=jax.ShapeDtypeStruct((N, NCLASS), jnp.float32),
        scratch_shapes=[
            pltpu.VMEM((N, NHID), jnp.float32),
            pltpu.VMEM((N, NCLASS), jnp.float32),
        ],
        compiler_params=pltpu.CompilerParams(
            dimension_semantics=("arbitrary",)),
    )(adj, x, W1, W2, b1.reshape(1, NHID), b2.reshape(1, NCLASS))


# speedup vs baseline: 1.0892x; 1.0157x over previous
import jax
import jax.numpy as jnp
from jax.experimental import pallas as pl
from jax.experimental.pallas import tpu as pltpu

N = 10000
NFEAT = 128
NHID = 16
NCLASS = 8

TI = 400
NI = N // TI
NC = 2
NSTEPS = 1 + 2 * NI


def _adj_index(g):
    return (jnp.where(g == 0, 0,
                      jnp.where(g <= NI, g - 1,
                                jnp.where(g <= NI + 1 + NC, NI - 1,
                                          g - NI - 2))), 0)


def _out_index(g):
    return (jnp.where(g <= NI + 1, NI - 1, g - NI - 2), 0)


def _log_softmax(z):
    m = jnp.max(z, axis=1, keepdims=True)
    return z - (jnp.log(jnp.sum(jnp.exp(z - m), axis=1, keepdims=True)) + m)


def _body(adj_ref, x_ref, w1_ref, w2_ref, b1_ref, b2_ref, o_ref,
          s1_ref, s2_ref, cache_ref):
    g = pl.program_id(0)

    @pl.when(g == 0)
    def _():
        s1_ref[...] = jnp.dot(x_ref[...], w1_ref[...],
                              preferred_element_type=jnp.float32)

    @pl.when((g >= 1) & (g <= NI))
    def _():
        h = jnp.maximum(jnp.dot(adj_ref[...], s1_ref[...],
                                preferred_element_type=jnp.float32)
                        + b1_ref[...], 0.0)
        s2_ref[pl.ds((g - 1) * TI, TI), :] = jnp.dot(
            h, w2_ref[...], preferred_element_type=jnp.float32)

        @pl.when(g <= NC)
        def _():
            cache_ref[pl.ds((g - 1) * TI, TI), :] = adj_ref[...].astype(
                jnp.bfloat16)

    @pl.when(g > NI)
    def _():
        use_cache = (g >= NI + 2) & (g <= NI + 1 + NC)

        @pl.when(use_cache)
        def _():
            blk = cache_ref[pl.ds((g - NI - 2) * TI, TI), :]
            o_ref[...] = _log_softmax(
                jnp.dot(blk, s2_ref[...].astype(jnp.bfloat16),
                        preferred_element_type=jnp.float32) + b2_ref[...])

        @pl.when(jnp.logical_not(use_cache))
        def _():
            o_ref[...] = _log_softmax(
                jnp.dot(adj_ref[...], s2_ref[...],
                        preferred_element_type=jnp.float32) + b2_ref[...])


def kernel(x, adj, W1, b1, W2, b2):
    return pl.pallas_call(
        _body,
        grid=(NSTEPS,),
        in_specs=[
            pl.BlockSpec((TI, N), _adj_index),
            pl.BlockSpec((N, NFEAT), lambda g: (0, 0)),
            pl.BlockSpec((NFEAT, NHID), lambda g: (0, 0)),
            pl.BlockSpec((NHID, NCLASS), lambda g: (0, 0)),
            pl.BlockSpec((1, NHID), lambda g: (0, 0)),
            pl.BlockSpec((1, NCLASS), lambda g: (0, 0)),
        ],
        out_specs=pl.BlockSpec((TI, NCLASS), _out_index),
        out_shape=jax.ShapeDtypeStruct((N, NCLASS), jnp.float32),
        scratch_shapes=[
            pltpu.VMEM((N, NHID), jnp.float32),
            pltpu.VMEM((N, NCLASS), jnp.float32),
            pltpu.VMEM((NC * TI, N), jnp.bfloat16),
        ],
        compiler_params=pltpu.CompilerParams(
            dimension_semantics=("arbitrary",),
            vmem_limit_bytes=100 * 1024 * 1024),
    )(adj, x, W1, W2, b1.reshape(1, NHID), b2.reshape(1, NCLASS))
